# baseline (speedup 1.0000x reference)
import functools

import jax
import jax.numpy as jnp
import numpy as np
from jax import lax
from jax.experimental import pallas as pl
from jax.experimental.pallas import tpu as pltpu
from jax.experimental.pallas import tpu_sc as plsc

N_NODES = 10000
MUL = 32
CUTOFF = 0.08
NB = 3
RH = 200
NNORM = 20.0
D = MUL * 4
NW = MUL * 5
MAX_EDGES = N_NODES * 32

EB = 2000
RPT = 384

N_TILES = 32
EPT = MAX_EDGES // N_TILES
SPT = -(-N_NODES // N_TILES)
NVJ = N_NODES // 16

_PERM_X = np.array([m * 4 + c for c in range(4) for m in range(MUL)], dtype=np.int32)
_PERM_W = np.array([m * 5 + k for k in range(5) for m in range(MUL)], dtype=np.int32)

_ISQ3 = float(1.0 / np.sqrt(3.0))
_ISQ2 = float(1.0 / np.sqrt(2.0))
_SHC = float(1.0 / np.sqrt(NNORM))
_SQ3 = float(np.sqrt(3.0))


def _swish(x):
    return x * jax.nn.sigmoid(x)


def _edge_builder_body(px_hbm, py_hbm, pz_hbm, lo_hbm, hi_hbm, rows_hbm, cols_hbm,
                       px, py, pz, lobuf, hibuf, rbuf, cbuf):
    wid = lax.axis_index("c") * 16 + lax.axis_index("s")
    pltpu.sync_copy(px_hbm, px)
    pltpu.sync_copy(py_hbm, py)
    pltpu.sync_copy(pz_hbm, pz)
    pltpu.sync_copy(lo_hbm.at[pl.ds(wid * SPT * 16, SPT * 16)], lobuf)
    pltpu.sync_copy(hi_hbm.at[pl.ds(wid * SPT * 16, SPT * 16)], hibuf)

    pad_r = jnp.zeros((16,), jnp.int32)
    pad_c = jnp.full((16,), N_NODES, jnp.int32)

    def prefill(k, _):
        rbuf[pl.ds(k * 16, 16)] = pad_r
        cbuf[pl.ds(k * 16, 16)] = pad_c
        return 0

    lax.fori_loop(0, (EPT + 16) // 16, prefill, 0)

    base_i = wid * SPT
    n_i = jnp.minimum(SPT, N_NODES - base_i)
    lanes = lax.iota(jnp.int32, 16)
    cut2 = jnp.float32(CUTOFF * CUTOFF)

    def per_source(li, off):
        i = base_i + li
        vb = (i // 16) * 16
        onehot = (lanes == (i - vb)).astype(jnp.float32)
        sx = jnp.sum(px[pl.ds(vb, 16)] * onehot)
        sy = jnp.sum(py[pl.ds(vb, 16)] * onehot)
        sz = jnp.sum(pz[pl.ds(vb, 16)] * onehot)
        rvec = jnp.full((16,), i, jnp.int32)
        b0 = lobuf[pl.ds(li * 16, 16)][0] // 16
        b1 = (hibuf[pl.ds(li * 16, 16)][0] + 15) // 16

        def per_block(b, off):
            dx = px[pl.ds(b * 16, 16)] - sx
            dy = py[pl.ds(b * 16, 16)] - sy
            dz = pz[pl.ds(b * 16, 16)] - sz
            d2 = dx * dx + dy * dy + dz * dz
            jvec = lanes + b * 16
            mask = (d2 < cut2) & (jvec != i)
            plsc.store_compressed(rbuf.at[pl.ds(off, 16)], jvec, mask=mask)
            plsc.store_compressed(cbuf.at[pl.ds(off, 16)], rvec, mask=mask)
            cnt = plsc.all_reduce_population_count(mask)[0]
            return jnp.minimum(off + cnt, EPT)

        return lax.fori_loop(b0, b1, per_block, off)

    lax.fori_loop(0, n_i, per_source, jnp.int32(0))

    pltpu.sync_copy(rbuf.at[pl.ds(0, EPT)], rows_hbm.at[pl.ds(wid * EPT, EPT)])
    pltpu.sync_copy(cbuf.at[pl.ds(0, EPT)], cols_hbm.at[pl.ds(wid * EPT, EPT)])


def _build_graph_sc(pos):
    mesh = plsc.VectorSubcoreMesh(core_axis_name="c", subcore_axis_name="s")
    builder = functools.partial(
        pl.kernel,
        mesh=mesh,
        compiler_params=pltpu.CompilerParams(needs_layout_passes=False),
        out_type=[
            jax.ShapeDtypeStruct((MAX_EDGES,), jnp.int32),
            jax.ShapeDtypeStruct((MAX_EDGES,), jnp.int32),
        ],
        scratch_types=[
            pltpu.VMEM((N_NODES,), jnp.float32),
            pltpu.VMEM((N_NODES,), jnp.float32),
            pltpu.VMEM((N_NODES,), jnp.float32),
            pltpu.VMEM((SPT * 16,), jnp.int32),
            pltpu.VMEM((SPT * 16,), jnp.int32),
            pltpu.VMEM((EPT + 16,), jnp.int32),
            pltpu.VMEM((EPT + 16,), jnp.int32),
        ],
    )(_edge_builder_body)
    xs = pos[:, 0]
    lo = jnp.searchsorted(xs, xs - CUTOFF, side='left').astype(jnp.int32)
    hi = jnp.searchsorted(xs, xs + CUTOFF, side='right').astype(jnp.int32)
    pad = N_TILES * SPT
    lo16 = jnp.broadcast_to(
        jnp.zeros((pad,), jnp.int32).at[:N_NODES].set(lo)[:, None],
        (pad, 16)).reshape(-1)
    hi16 = jnp.broadcast_to(
        jnp.zeros((pad,), jnp.int32).at[:N_NODES].set(hi)[:, None],
        (pad, 16)).reshape(-1)
    return builder(xs, pos[:, 1], pos[:, 2], lo16, hi16)


def _edge_block_kernel(ev_ref, xj_ref, col_ref, R1_ref, b1_ref, R2_ref, b2_ref,
                       R3_ref, b3_ref, agg_ref):
    ev = ev_ref[...]
    r2 = jnp.sum(ev * ev, axis=1, keepdims=True)
    r = jnp.sqrt(r2)
    inv = _SQ3 * _SHC / jnp.maximum(r, 1e-9)
    shv = ev * inv
    sh1 = shv[:, 0:1]
    sh2 = shv[:, 1:2]
    sh3 = shv[:, 2:3]

    step = CUTOFF / (NB - 1)
    mu = lax.broadcasted_iota(jnp.int32, (1, NB), 1).astype(jnp.float32) * step
    sig = 0.6 * step
    basis = jnp.exp(-((r - mu) ** 2) / (2.0 * sig * sig))

    h1 = jnp.dot(basis, R1_ref[...], preferred_element_type=jnp.float32) + b1_ref[...]
    h1 = _swish(h1)
    h2 = jnp.dot(h1, R2_ref[...], preferred_element_type=jnp.float32) + b2_ref[...]
    h2 = _swish(h2)
    w = jnp.dot(h2, R3_ref[...], preferred_element_type=jnp.float32) + b3_ref[...]
    W0 = w[:, 0 * D:1 * D]
    W1 = w[:, 1 * D:2 * D]
    W2 = w[:, 2 * D:3 * D]
    W3 = w[:, 3 * D:4 * D]

    xj = xj_ref[...].astype(jnp.float32)
    xr1 = pltpu.roll(xj, 3 * MUL, 1)
    xr2 = pltpu.roll(xj, 2 * MUL, 1)
    xr3 = pltpu.roll(xj, MUL, 1)

    gsel = (lax.broadcasted_iota(jnp.int32, (1, D), 1) // MUL) % 2 == 0
    p13 = jnp.where(gsel, sh1, sh3)
    p31 = jnp.where(gsel, sh3, sh1)

    m = (W0 * xj + (W1 * p13) * xr1 + W2 * (sh2 * xr2) + (W3 * p31) * xr3)

    i = pl.program_id(0)
    k = pl.program_id(1)
    t = col_ref[...] - i * SPT
    oh = (t == lax.broadcasted_iota(jnp.int32, (1, RPT), 1))
    partial = lax.dot_general(oh.astype(jnp.bfloat16), m.astype(jnp.bfloat16),
                              (((0,), (0,)), ((), ())),
                              preferred_element_type=jnp.float32)

    @pl.when(k == 0)
    def _():
        agg_ref[...] = partial[None]

    @pl.when(k > 0)
    def _():
        agg_ref[...] += partial[None]


def _edge_messages(ev, xj, col, R1, b1, R2, b2, R3, b3):
    full = lambda shape: pl.BlockSpec(shape, lambda i, k: (0, 0))
    agg = pl.pallas_call(
        _edge_block_kernel,
        grid=(N_TILES, EPT // EB),
        in_specs=[
            pl.BlockSpec((EB, 3), lambda i, k: (i * (EPT // EB) + k, 0)),
            pl.BlockSpec((EB, D), lambda i, k: (i * (EPT // EB) + k, 0)),
            pl.BlockSpec((EB, 1), lambda i, k: (i * (EPT // EB) + k, 0)),
            full((NB, RH)), full((1, RH)),
            full((RH, RH)), full((1, RH)),
            full((RH, 4 * D)), full((1, 4 * D)),
        ],
        out_specs=pl.BlockSpec((1, RPT, D), lambda i, k: (i, 0, 0)),
        out_shape=jax.ShapeDtypeStruct((N_TILES, RPT, D), jnp.float32),
        compiler_params=pltpu.CompilerParams(
            dimension_semantics=("parallel", "arbitrary")),
    )(ev, xj, col.reshape(MAX_EDGES, 1), R1, b1, R2, b2, R3, b3)
    return agg.reshape(N_TILES * RPT, D)


def _arrange_R3(R3, b3):
    idx = np.arange(MUL) * 5
    c = [R3[:, idx + k] for k in range(5)]
    cb = [b3[idx + k] for k in range(5)]
    blocks = [
        [c[0] * _SHC, c[2] * _SHC, c[2] * _SHC, c[2] * _SHC],
        [c[3] * _ISQ3, c[4] * _ISQ2, c[4] * _ISQ2, c[1]],
        [c[3] * _ISQ3, -c[4] * _ISQ2, c[1], c[4] * _ISQ2],
        [c[3] * _ISQ3, c[1], -c[4] * _ISQ2, -c[4] * _ISQ2],
    ]
    bblocks = [
        [cb[0] * _SHC, cb[2] * _SHC, cb[2] * _SHC, cb[2] * _SHC],
        [cb[3] * _ISQ3, cb[4] * _ISQ2, cb[4] * _ISQ2, cb[1]],
        [cb[3] * _ISQ3, -cb[4] * _ISQ2, cb[1], cb[4] * _ISQ2],
        [cb[3] * _ISQ3, cb[1], -cb[4] * _ISQ2, -cb[4] * _ISQ2],
    ]
    R3X = jnp.concatenate([jnp.concatenate(b, axis=1) for b in blocks], axis=1)
    b3X = jnp.concatenate([jnp.concatenate(b, axis=0) for b in bblocks], axis=0)
    return R3X, b3X


def _gate_cm(x):
    s = x[:, :MUL]
    v = x[:, MUL:]
    return jnp.concatenate(
        [_swish(s), jnp.tile(jax.nn.sigmoid(s), (1, 3)) * v], axis=1)


def kernel(z, pos, batch, emb,
           W1_0, W2_0, R1_0, b1_0, R2_0, b2_0, R3_0, b3_0,
           W1_1, W2_1, R1_1, b1_1, R2_1, b2_1, R3_1, b3_1,
           W1_2, W2_2, R1_2, b1_2, R2_2, b2_2, R3_2, b3_2):
    n = z.shape[0]
    order = jnp.argsort(pos[:, 0])
    pos = pos[order]
    z = z[order]
    row, col = _build_graph_sc(pos)
    edge_vec = pos[row] - pos[col]

    h = jnp.concatenate(
        [emb[z], jnp.zeros((n, 3 * MUL), emb.dtype)], axis=1)

    layers = [
        (W1_0, W2_0, R1_0, b1_0, R2_0, b2_0, R3_0, b3_0),
        (W1_1, W2_1, R1_1, b1_1, R2_1, b2_1, R3_1, b3_1),
        (W1_2, W2_2, R1_2, b1_2, R2_2, b2_2, R3_2, b3_2),
    ]
    for li, (W1, W2, R1, b1, R2, b2, R3, b3) in enumerate(layers):
        W1cm = W1[_PERM_X]
        W2cm = W2[_PERM_X]
        if W1.shape[1] == D:
            W1cm = W1cm[:, _PERM_X]
            W2cm = W2cm[:, _PERM_X]
        R3X, b3X = _arrange_R3(R3, b3)
        si = h @ W1cm
        xj = h.astype(jnp.bfloat16)[row]
        aggf = _edge_messages(edge_vec, xj, col,
                              R1, b1.reshape(1, RH), R2, b2.reshape(1, RH),
                              R3X, b3X.reshape(1, 4 * D))
        agg = aggf.reshape(N_TILES, RPT, D)[:, :SPT, :].reshape(N_TILES * SPT, D)[:n]
        h = (0.5 ** 0.5) * (si + agg @ W2cm)
        if li < 2:
            h = _gate_cm(h)

    return jnp.sum(h, axis=0).reshape(1, -1)


# speedup vs baseline: 1.1606x; 1.1606x over previous
import functools

import jax
import jax.numpy as jnp
import numpy as np
from jax import lax
from jax.experimental import pallas as pl
from jax.experimental.pallas import tpu as pltpu
from jax.experimental.pallas import tpu_sc as plsc

N_NODES = 10000
MUL = 32
CUTOFF = 0.08
NB = 3
RH = 200
NNORM = 20.0
D = MUL * 4
NW = MUL * 5
MAX_EDGES = N_NODES * 32

EB = 2000
RPT = 384

N_TILES = 32
EPT = MAX_EDGES // N_TILES
SPT = -(-N_NODES // N_TILES)
NVJ = N_NODES // 16

_PERM_X = np.array([m * 4 + c for c in range(4) for m in range(MUL)], dtype=np.int32)
_PERM_W = np.array([m * 5 + k for k in range(5) for m in range(MUL)], dtype=np.int32)

_ISQ3 = float(1.0 / np.sqrt(3.0))
_ISQ2 = float(1.0 / np.sqrt(2.0))
_SHC = float(1.0 / np.sqrt(NNORM))
_SQ3 = float(np.sqrt(3.0))


def _swish(x):
    return x * jax.nn.sigmoid(x)


def _edge_builder_body(px_hbm, py_hbm, pz_hbm, lo_hbm, hi_hbm, rows_hbm, cols_hbm,
                       evx_hbm, evy_hbm, evz_hbm,
                       px, py, pz, lobuf, hibuf, rbuf, cbuf, xbuf, ybuf, zbuf):
    wid = lax.axis_index("c") * 16 + lax.axis_index("s")
    pltpu.sync_copy(px_hbm, px)
    pltpu.sync_copy(py_hbm, py)
    pltpu.sync_copy(pz_hbm, pz)
    pltpu.sync_copy(lo_hbm.at[pl.ds(wid * SPT * 16, SPT * 16)], lobuf)
    pltpu.sync_copy(hi_hbm.at[pl.ds(wid * SPT * 16, SPT * 16)], hibuf)

    pad_r = jnp.zeros((16,), jnp.int32)
    pad_c = jnp.full((16,), N_NODES, jnp.int32)

    pad_z = jnp.zeros((16,), jnp.float32)

    def prefill(k, _):
        rbuf[pl.ds(k * 16, 16)] = pad_r
        cbuf[pl.ds(k * 16, 16)] = pad_c
        xbuf[pl.ds(k * 16, 16)] = pad_z
        ybuf[pl.ds(k * 16, 16)] = pad_z
        zbuf[pl.ds(k * 16, 16)] = pad_z
        return 0

    lax.fori_loop(0, (EPT + 16) // 16, prefill, 0)

    base_i = wid * SPT
    n_i = jnp.minimum(SPT, N_NODES - base_i)
    lanes = lax.iota(jnp.int32, 16)
    cut2 = jnp.float32(CUTOFF * CUTOFF)

    def per_source(li, off):
        i = base_i + li
        vb = (i // 16) * 16
        onehot = (lanes == (i - vb)).astype(jnp.float32)
        sx = jnp.sum(px[pl.ds(vb, 16)] * onehot)
        sy = jnp.sum(py[pl.ds(vb, 16)] * onehot)
        sz = jnp.sum(pz[pl.ds(vb, 16)] * onehot)
        rvec = jnp.full((16,), i, jnp.int32)
        b0 = lobuf[pl.ds(li * 16, 16)][0] // 16
        b1 = (hibuf[pl.ds(li * 16, 16)][0] + 15) // 16

        def per_block(b, off):
            dx = px[pl.ds(b * 16, 16)] - sx
            dy = py[pl.ds(b * 16, 16)] - sy
            dz = pz[pl.ds(b * 16, 16)] - sz
            d2 = dx * dx + dy * dy + dz * dz
            jvec = lanes + b * 16
            mask = (d2 < cut2) & (jvec != i)
            plsc.store_compressed(rbuf.at[pl.ds(off, 16)], jvec, mask=mask)
            plsc.store_compressed(cbuf.at[pl.ds(off, 16)], rvec, mask=mask)
            plsc.store_compressed(xbuf.at[pl.ds(off, 16)], dx, mask=mask)
            plsc.store_compressed(ybuf.at[pl.ds(off, 16)], dy, mask=mask)
            plsc.store_compressed(zbuf.at[pl.ds(off, 16)], dz, mask=mask)
            cnt = plsc.all_reduce_population_count(mask)[0]
            return jnp.minimum(off + cnt, EPT)

        return lax.fori_loop(b0, b1, per_block, off)

    lax.fori_loop(0, n_i, per_source, jnp.int32(0))

    pltpu.sync_copy(rbuf.at[pl.ds(0, EPT)], rows_hbm.at[pl.ds(wid * EPT, EPT)])
    pltpu.sync_copy(cbuf.at[pl.ds(0, EPT)], cols_hbm.at[pl.ds(wid * EPT, EPT)])
    pltpu.sync_copy(xbuf.at[pl.ds(0, EPT)], evx_hbm.at[pl.ds(wid * EPT, EPT)])
    pltpu.sync_copy(ybuf.at[pl.ds(0, EPT)], evy_hbm.at[pl.ds(wid * EPT, EPT)])
    pltpu.sync_copy(zbuf.at[pl.ds(0, EPT)], evz_hbm.at[pl.ds(wid * EPT, EPT)])


def _build_graph_sc(pos):
    mesh = plsc.VectorSubcoreMesh(core_axis_name="c", subcore_axis_name="s")
    builder = functools.partial(
        pl.kernel,
        mesh=mesh,
        compiler_params=pltpu.CompilerParams(needs_layout_passes=False),
        out_type=[
            jax.ShapeDtypeStruct((MAX_EDGES,), jnp.int32),
            jax.ShapeDtypeStruct((MAX_EDGES,), jnp.int32),
            jax.ShapeDtypeStruct((MAX_EDGES,), jnp.float32),
            jax.ShapeDtypeStruct((MAX_EDGES,), jnp.float32),
            jax.ShapeDtypeStruct((MAX_EDGES,), jnp.float32),
        ],
        scratch_types=[
            pltpu.VMEM((N_NODES,), jnp.float32),
            pltpu.VMEM((N_NODES,), jnp.float32),
            pltpu.VMEM((N_NODES,), jnp.float32),
            pltpu.VMEM((SPT * 16,), jnp.int32),
            pltpu.VMEM((SPT * 16,), jnp.int32),
            pltpu.VMEM((EPT + 16,), jnp.int32),
            pltpu.VMEM((EPT + 16,), jnp.int32),
            pltpu.VMEM((EPT + 16,), jnp.float32),
            pltpu.VMEM((EPT + 16,), jnp.float32),
            pltpu.VMEM((EPT + 16,), jnp.float32),
        ],
    )(_edge_builder_body)
    xs = pos[:, 0]
    lo = jnp.searchsorted(xs, xs - CUTOFF, side='left').astype(jnp.int32)
    hi = jnp.searchsorted(xs, xs + CUTOFF, side='right').astype(jnp.int32)
    pad = N_TILES * SPT
    lo16 = jnp.broadcast_to(
        jnp.zeros((pad,), jnp.int32).at[:N_NODES].set(lo)[:, None],
        (pad, 16)).reshape(-1)
    hi16 = jnp.broadcast_to(
        jnp.zeros((pad,), jnp.int32).at[:N_NODES].set(hi)[:, None],
        (pad, 16)).reshape(-1)
    return builder(xs, pos[:, 1], pos[:, 2], lo16, hi16)


def _edge_block_kernel(evx_ref, evy_ref, evz_ref, xj_ref, col_ref,
                       R1_ref, b1_ref, R2_ref, b2_ref,
                       R3_ref, b3_ref, agg_ref):
    dx = evx_ref[...]
    dy = evy_ref[...]
    dz = evz_ref[...]
    r2 = dx * dx + dy * dy + dz * dz
    r = jnp.sqrt(r2)
    inv = _SQ3 * _SHC / jnp.maximum(r, 1e-9)
    sh1 = dx * inv
    sh2 = dy * inv
    sh3 = dz * inv

    step = CUTOFF / (NB - 1)
    mu = lax.broadcasted_iota(jnp.int32, (1, NB), 1).astype(jnp.float32) * step
    sig = 0.6 * step
    basis = jnp.exp(-((r - mu) ** 2) / (2.0 * sig * sig))

    h1 = jnp.dot(basis, R1_ref[...], preferred_element_type=jnp.float32) + b1_ref[...]
    h1 = _swish(h1)
    h2 = jnp.dot(h1, R2_ref[...], preferred_element_type=jnp.float32) + b2_ref[...]
    h2 = _swish(h2)
    w = jnp.dot(h2, R3_ref[...], preferred_element_type=jnp.float32) + b3_ref[...]
    W0 = w[:, 0 * D:1 * D]
    W1 = w[:, 1 * D:2 * D]
    W2 = w[:, 2 * D:3 * D]
    W3 = w[:, 3 * D:4 * D]

    xj = xj_ref[...].astype(jnp.float32)
    xr1 = pltpu.roll(xj, 3 * MUL, 1)
    xr2 = pltpu.roll(xj, 2 * MUL, 1)
    xr3 = pltpu.roll(xj, MUL, 1)

    gsel = (lax.broadcasted_iota(jnp.int32, (1, D), 1) // MUL) % 2 == 0
    p13 = jnp.where(gsel, sh1, sh3)
    p31 = jnp.where(gsel, sh3, sh1)

    m = (W0 * xj + (W1 * p13) * xr1 + W2 * (sh2 * xr2) + (W3 * p31) * xr3)

    i = pl.program_id(0)
    k = pl.program_id(1)
    t = col_ref[...] - i * SPT
    oh = (t == lax.broadcasted_iota(jnp.int32, (1, RPT), 1))
    partial = lax.dot_general(oh.astype(jnp.bfloat16), m.astype(jnp.bfloat16),
                              (((0,), (0,)), ((), ())),
                              preferred_element_type=jnp.float32)

    @pl.when(k == 0)
    def _():
        agg_ref[...] = partial[None]

    @pl.when(k > 0)
    def _():
        agg_ref[...] += partial[None]


def _edge_messages(evx, evy, evz, xj, col, R1, b1, R2, b2, R3, b3):
    full = lambda shape: pl.BlockSpec(shape, lambda i, k: (0, 0))
    agg = pl.pallas_call(
        _edge_block_kernel,
        grid=(N_TILES, EPT // EB),
        in_specs=[
            pl.BlockSpec((EB, 1), lambda i, k: (i * (EPT // EB) + k, 0)),
            pl.BlockSpec((EB, 1), lambda i, k: (i * (EPT // EB) + k, 0)),
            pl.BlockSpec((EB, 1), lambda i, k: (i * (EPT // EB) + k, 0)),
            pl.BlockSpec((EB, D), lambda i, k: (i * (EPT // EB) + k, 0)),
            pl.BlockSpec((EB, 1), lambda i, k: (i * (EPT // EB) + k, 0)),
            full((NB, RH)), full((1, RH)),
            full((RH, RH)), full((1, RH)),
            full((RH, 4 * D)), full((1, 4 * D)),
        ],
        out_specs=pl.BlockSpec((1, RPT, D), lambda i, k: (i, 0, 0)),
        out_shape=jax.ShapeDtypeStruct((N_TILES, RPT, D), jnp.float32),
        compiler_params=pltpu.CompilerParams(
            dimension_semantics=("parallel", "arbitrary")),
    )(evx.reshape(MAX_EDGES, 1), evy.reshape(MAX_EDGES, 1),
      evz.reshape(MAX_EDGES, 1), xj, col.reshape(MAX_EDGES, 1),
      R1, b1, R2, b2, R3, b3)
    return agg.reshape(N_TILES * RPT, D)


def _arrange_R3(R3, b3):
    idx = np.arange(MUL) * 5
    c = [R3[:, idx + k] for k in range(5)]
    cb = [b3[idx + k] for k in range(5)]
    blocks = [
        [c[0] * _SHC, c[2] * _SHC, c[2] * _SHC, c[2] * _SHC],
        [c[3] * _ISQ3, c[4] * _ISQ2, c[4] * _ISQ2, c[1]],
        [c[3] * _ISQ3, -c[4] * _ISQ2, c[1], c[4] * _ISQ2],
        [c[3] * _ISQ3, c[1], -c[4] * _ISQ2, -c[4] * _ISQ2],
    ]
    bblocks = [
        [cb[0] * _SHC, cb[2] * _SHC, cb[2] * _SHC, cb[2] * _SHC],
        [cb[3] * _ISQ3, cb[4] * _ISQ2, cb[4] * _ISQ2, cb[1]],
        [cb[3] * _ISQ3, -cb[4] * _ISQ2, cb[1], cb[4] * _ISQ2],
        [cb[3] * _ISQ3, cb[1], -cb[4] * _ISQ2, -cb[4] * _ISQ2],
    ]
    R3X = jnp.concatenate([jnp.concatenate(b, axis=1) for b in blocks], axis=1)
    b3X = jnp.concatenate([jnp.concatenate(b, axis=0) for b in bblocks], axis=0)
    return R3X, b3X


def _gate_cm(x):
    s = x[:, :MUL]
    v = x[:, MUL:]
    return jnp.concatenate(
        [_swish(s), jnp.tile(jax.nn.sigmoid(s), (1, 3)) * v], axis=1)


def kernel(z, pos, batch, emb,
           W1_0, W2_0, R1_0, b1_0, R2_0, b2_0, R3_0, b3_0,
           W1_1, W2_1, R1_1, b1_1, R2_1, b2_1, R3_1, b3_1,
           W1_2, W2_2, R1_2, b1_2, R2_2, b2_2, R3_2, b3_2):
    n = z.shape[0]
    order = jnp.argsort(pos[:, 0])
    pos = pos[order]
    z = z[order]
    row, col, evx, evy, evz = _build_graph_sc(pos)

    h = jnp.concatenate(
        [emb[z], jnp.zeros((n, 3 * MUL), emb.dtype)], axis=1)

    layers = [
        (W1_0, W2_0, R1_0, b1_0, R2_0, b2_0, R3_0, b3_0),
        (W1_1, W2_1, R1_1, b1_1, R2_1, b2_1, R3_1, b3_1),
        (W1_2, W2_2, R1_2, b1_2, R2_2, b2_2, R3_2, b3_2),
    ]
    for li, (W1, W2, R1, b1, R2, b2, R3, b3) in enumerate(layers):
        W1cm = W1[_PERM_X]
        W2cm = W2[_PERM_X]
        if W1.shape[1] == D:
            W1cm = W1cm[:, _PERM_X]
            W2cm = W2cm[:, _PERM_X]
        R3X, b3X = _arrange_R3(R3, b3)
        si = h @ W1cm
        xj = h.astype(jnp.bfloat16)[row]
        aggf = _edge_messages(evx, evy, evz, xj, col,
                              R1, b1.reshape(1, RH), R2, b2.reshape(1, RH),
                              R3X, b3X.reshape(1, 4 * D))
        agg = aggf.reshape(N_TILES, RPT, D)[:, :SPT, :].reshape(N_TILES * SPT, D)[:n]
        h = (0.5 ** 0.5) * (si + agg @ W2cm)
        if li < 2:
            h = _gate_cm(h)

    return jnp.sum(h, axis=0).reshape(1, -1)
